# baseline (speedup 1.0000x reference)
import jax
import jax.numpy as jnp
from jax import lax
from jax.experimental import pallas as pl
from jax.experimental.pallas import tpu as pltpu
from jax.experimental.pallas import tpu_sc as plsc

N = 100000
D = 128
S = 64
NC = 2
NS = 16
L = 16
DH = D // NC
BLK = 448
Q = 3136
N_SC = NS * Q
NBLK = Q // BLK
G = 32
ACC_ROWS = 80
TC_BLOCKS = (N - N_SC + BLK - 1) // BLK


def _first_lane(v):
    return lax.squeeze(lax.slice(v, (0,), (1,)), (0,))


def _last_lane(v):
    return lax.squeeze(lax.slice(v, (L - 1,), (L,)), (0,))


def _tree(vs):
    while len(vs) > 1:
        vs = [vs[i] + vs[i + 1] for i in range(0, len(vs) - 1, 2)] \
             + ([vs[-1]] if len(vs) % 2 else [])
    return vs[0]


def _pool_body(x_hbm, b_hbm, sum_hbm, cnt_hbm,
               xbig, idxbig, idbuf, zbuf,
               acc_local, cnt_local, acc_sh, cnt_sh, sx, si):
    cid = lax.axis_index("c")
    sid = lax.axis_index("s")
    col0 = cid * DH
    base0 = sid * Q

    zero16 = jnp.zeros((L,), jnp.float32)

    def _zrow(r, carry):
        for l in range(DH // L):
            acc_local[r, pl.ds(l * L, L)] = zero16
        cnt_local[r, pl.ds(0, L)] = zero16
        return carry
    lax.fori_loop(0, ACC_ROWS, _zrow, 0)

    iota16 = lax.iota(jnp.int32, 16)
    for k in range(ACC_ROWS // 16):
        idbuf[0, pl.ds(k * 16, 16)] = iota16 + (k * 16)

    for r in range(16):
        for l in range(DH // L):
            zbuf[r, pl.ds(l * L, L)] = zero16
    @pl.when(sid == 0)
    def _():
        for r0 in range(0, ACC_ROWS, 16):
            pltpu.sync_copy(zbuf, acc_sh.at[pl.ds(r0, 16)])
            pltpu.sync_copy(zbuf.at[:, pl.ds(0, 16)], cnt_sh.at[pl.ds(r0, 16)])

    def _x_slices(j):
        base = base0 + j * BLK
        sel = lax.rem(j, 2)
        return (x_hbm.at[pl.ds(base, BLK), pl.ds(col0, DH)],
                xbig.at[pl.ds(sel * BLK, BLK)])

    def _i_slices(j):
        base = base0 + j * BLK
        sel = lax.rem(j, 2)
        return b_hbm.at[pl.ds(base, BLK)], idxbig.at[sel]

    def _rows(sel, g, idxvs):
        rb = sel * BLK + g * G
        for v in range(G // L):
            for r in range(L):
                sr = lax.squeeze(lax.slice(idxvs[v], (r,), (r + 1,)), (0,))
                for l in range(DH // L):
                    acc_local[sr, pl.ds(l * L, L)] = (
                        acc_local[sr, pl.ds(l * L, L)]
                        + xbig[rb + v * L + r, pl.ds(l * L, L)])
                cnt_local[sr, pl.ds(0, L)] = cnt_local[sr, pl.ds(0, L)] + 1.0

    def _group(sel, g):
        rb = sel * BLK + g * G
        idxvs = [idxbig[sel, pl.ds(g * G + v * L, L)] for v in range(G // L)]
        seg0 = _first_lane(idxvs[0])
        uniform = seg0 == _last_lane(idxvs[-1])

        @pl.when(uniform)
        def _():
            for l in range(DH // L):
                s = _tree([xbig[rb + r, pl.ds(l * L, L)] for r in range(G)])
                acc_local[seg0, pl.ds(l * L, L)] = (
                    acc_local[seg0, pl.ds(l * L, L)] + s)
            cnt_local[seg0, pl.ds(0, L)] = cnt_local[seg0, pl.ds(0, L)] + float(G)

        @pl.when(jnp.logical_not(uniform))
        def _():
            _rows(sel, g, idxvs)

    def _block(j, carry):
        pltpu.make_async_copy(*_x_slices(j), sx).wait()
        pltpu.make_async_copy(*_i_slices(j), si).wait()

        @pl.when(j + 1 < NBLK)
        def _():
            pltpu.async_copy(*_x_slices(j + 1), sx)
            pltpu.async_copy(*_i_slices(j + 1), si)

        sel = lax.rem(j, 2)
        segb = _first_lane(idxbig[sel, pl.ds(0, L)])
        segl = _last_lane(idxbig[sel, pl.ds(BLK - L, L)])
        uniform_block = segb == segl

        @pl.when(uniform_block)
        def _():
            def _acc32(g, c):
                rb = sel * BLK + g * G
                parts = []
                for l in range(DH // L):
                    parts.append(_tree(
                        [xbig[rb + r, pl.ds(l * L, L)] for r in range(G)]))
                return tuple(c[l] + parts[l] for l in range(DH // L))
            tot = lax.fori_loop(0, BLK // G,
                                _acc32, tuple([zero16] * (DH // L)))
            for l in range(DH // L):
                acc_local[segb, pl.ds(l * L, L)] = (
                    acc_local[segb, pl.ds(l * L, L)] + tot[l])
            cnt_local[segb, pl.ds(0, L)] = (
                cnt_local[segb, pl.ds(0, L)] + float(BLK))

        @pl.when(jnp.logical_not(uniform_block))
        def _():
            def _g(g, c):
                _group(sel, g)
                return c
            lax.fori_loop(0, BLK // G, _g, 0)
        return carry
    pltpu.async_copy(*_x_slices(0), sx)
    pltpu.async_copy(*_i_slices(0), si)
    lax.fori_loop(0, NBLK, _block, 0)

    plsc.subcore_barrier()

    idrow = idbuf.at[0]
    pltpu.sync_copy(acc_local, acc_sh.at[idrow], add=True)
    pltpu.sync_copy(cnt_local, cnt_sh.at[idrow], add=True)

    plsc.subcore_barrier()

    @pl.when(sid < ACC_ROWS // 16)
    def _():
        r0 = sid * 16
        pltpu.sync_copy(acc_sh.at[pl.ds(r0, 16)],
                        sum_hbm.at[pl.ds(r0, 16), pl.ds(col0, DH)])
        @pl.when(cid == 0)
        def _():
            pltpu.sync_copy(cnt_sh.at[pl.ds(r0, 16)], cnt_hbm.at[pl.ds(r0, 16)])


_mesh = plsc.VectorSubcoreMesh(core_axis_name="c", subcore_axis_name="s",
                               num_cores=NC, num_subcores=NS)

_pool_sc = pl.kernel(
    _pool_body,
    out_type=(jax.ShapeDtypeStruct((ACC_ROWS, D), jnp.float32),
              jax.ShapeDtypeStruct((ACC_ROWS, 16), jnp.float32)),
    mesh=_mesh,
    scratch_types=[
        pltpu.VMEM((2 * BLK, DH), jnp.float32),
        pltpu.VMEM((2, BLK), jnp.int32),
        pltpu.VMEM((1, ACC_ROWS), jnp.int32),
        pltpu.VMEM((16, DH), jnp.float32),
        pltpu.VMEM((ACC_ROWS, DH), jnp.float32),
        pltpu.VMEM((ACC_ROWS, 16), jnp.float32),
        pltpu.VMEM_SHARED((ACC_ROWS, DH), jnp.float32),
        pltpu.VMEM_SHARED((ACC_ROWS, 16), jnp.float32),
        pltpu.SemaphoreType.DMA,
        pltpu.SemaphoreType.DMA,
    ],
    compiler_params=pltpu.CompilerParams(use_tc_tiling_on_sc=False,
                                         needs_layout_passes=False),
)


def _tc_body(x_ref, b_ref, sum_ref, cnt_ref):
    i = pl.program_id(0)
    b = b_ref[0, 0, :]
    seg_ids = lax.broadcasted_iota(jnp.int32, (BLK, S), 1)
    oh = (b[:, None] == seg_ids).astype(jnp.float32)
    row = (N_SC + i * BLK) + lax.broadcasted_iota(jnp.int32, (BLK, 1), 0)
    xm = jnp.where(row < N, x_ref[...], 0.0)
    part = lax.dot_general(oh, xm, (((0,), (0,)), ((), ())),
                           preferred_element_type=jnp.float32)
    pcnt = lax.dot_general(oh, jnp.ones((BLK, 8), jnp.float32),
                           (((0,), (0,)), ((), ())),
                           preferred_element_type=jnp.float32)

    @pl.when(i == 0)
    def _():
        sum_ref[...] = part
        cnt_ref[...] = pcnt

    @pl.when(i > 0)
    def _():
        sum_ref[...] = sum_ref[...] + part
        cnt_ref[...] = cnt_ref[...] + pcnt


_pool_tc = pl.pallas_call(
    _tc_body,
    grid=(TC_BLOCKS,),
    in_specs=[
        pl.BlockSpec((BLK, D), lambda i: (N_SC // BLK + i, 0)),
        pl.BlockSpec((1, 1, BLK), lambda i: (i, 0, 0)),
    ],
    out_specs=[
        pl.BlockSpec((S, D), lambda i: (0, 0)),
        pl.BlockSpec((S, 8), lambda i: (0, 0)),
    ],
    out_shape=[
        jax.ShapeDtypeStruct((S, D), jnp.float32),
        jax.ShapeDtypeStruct((S, 8), jnp.float32),
    ],
)


def _comb_body(ssc_ref, csc_ref, stc_ref, ctc_ref, out_ref):
    c = csc_ref[...][0:S, 0:1] + ctc_ref[...][0:S, 0:1]
    out_ref[...] = ((ssc_ref[...][0:S, :] + stc_ref[...])
                    / jnp.maximum(c, 1.0))


_combine = pl.pallas_call(
    _comb_body,
    out_shape=jax.ShapeDtypeStruct((S, D), jnp.float32),
)


def kernel(x, batch):
    b32 = batch.astype(jnp.int32)
    sum_sc, cnt_sc = _pool_sc(x, b32)
    b_tc = jnp.pad(b32[N_SC:], (0, TC_BLOCKS * BLK - (N - N_SC)),
                   constant_values=S).reshape(TC_BLOCKS, 1, BLK)
    sum_tc, cnt_tc = _pool_tc(x, b_tc)
    return _combine(sum_sc, cnt_sc, sum_tc, cnt_tc)


# speedup vs baseline: 1.6515x; 1.6515x over previous
import jax
import jax.numpy as jnp
from jax import lax
from jax.experimental import pallas as pl
from jax.experimental.pallas import tpu as pltpu
from jax.experimental.pallas import tpu_sc as plsc

N = 100000
D = 128
S = 64
NC = 2
NS = 16
L = 16
DH = D // NC
BLK = 448
Q = 3136
N_SC = NS * Q
NBLK = Q // BLK
G = 32
ACC_ROWS = 80
B_TC = 896
TC_END = 99456
TC_BLOCKS = (TC_END - N_SC) // B_TC
TAIL = N - TC_END


def _first_lane(v):
    return lax.squeeze(lax.slice(v, (0,), (1,)), (0,))


def _last_lane(v):
    return lax.squeeze(lax.slice(v, (L - 1,), (L,)), (0,))


def _tree(vs):
    while len(vs) > 1:
        vs = [vs[i] + vs[i + 1] for i in range(0, len(vs) - 1, 2)] \
             + ([vs[-1]] if len(vs) % 2 else [])
    return vs[0]


def _pool_body(x_hbm, b_hbm, sum_hbm, cnt_hbm,
               xbig, idxbig, idbuf, zbuf,
               acc_local, cnt_local, acc_sh, cnt_sh, sx, si):
    cid = lax.axis_index("c")
    sid = lax.axis_index("s")
    col0 = cid * DH
    base0 = sid * Q

    zero16 = jnp.zeros((L,), jnp.float32)

    def _zrow(r, carry):
        for l in range(DH // L):
            acc_local[r, pl.ds(l * L, L)] = zero16
        cnt_local[r, pl.ds(0, L)] = zero16
        return carry
    lax.fori_loop(0, ACC_ROWS, _zrow, 0)

    iota16 = lax.iota(jnp.int32, 16)
    for k in range(ACC_ROWS // 16):
        idbuf[0, pl.ds(k * 16, 16)] = iota16 + (k * 16)

    for r in range(16):
        for l in range(DH // L):
            zbuf[r, pl.ds(l * L, L)] = zero16
    @pl.when(sid == 0)
    def _():
        for r0 in range(0, ACC_ROWS, 16):
            pltpu.sync_copy(zbuf, acc_sh.at[pl.ds(r0, 16)])
            pltpu.sync_copy(zbuf.at[:, pl.ds(0, 16)], cnt_sh.at[pl.ds(r0, 16)])

    def _x_slices(j):
        base = base0 + j * BLK
        sel = lax.rem(j, 2)
        return (x_hbm.at[pl.ds(base, BLK), pl.ds(col0, DH)],
                xbig.at[pl.ds(sel * BLK, BLK)])

    def _i_slices(j):
        base = base0 + j * BLK
        sel = lax.rem(j, 2)
        return b_hbm.at[pl.ds(base, BLK)], idxbig.at[sel, pl.ds(0, BLK)]

    def _rows(sel, g, idxvs):
        rb = sel * BLK + g * G
        for v in range(G // L):
            for r in range(L):
                sr = lax.squeeze(lax.slice(idxvs[v], (r,), (r + 1,)), (0,))
                for l in range(DH // L):
                    acc_local[sr, pl.ds(l * L, L)] = (
                        acc_local[sr, pl.ds(l * L, L)]
                        + xbig[rb + v * L + r, pl.ds(l * L, L)])
                cnt_local[sr, pl.ds(0, L)] = cnt_local[sr, pl.ds(0, L)] + 1.0

    def _group(sel, g):
        rb = sel * BLK + g * G
        idxvs = [idxbig[sel, pl.ds(g * G + v * L, L)] for v in range(G // L)]
        seg0 = _first_lane(idxvs[0])
        uniform = seg0 == _last_lane(idxvs[-1])

        @pl.when(uniform)
        def _():
            for l in range(DH // L):
                s = _tree([xbig[rb + r, pl.ds(l * L, L)] for r in range(G)])
                acc_local[seg0, pl.ds(l * L, L)] = (
                    acc_local[seg0, pl.ds(l * L, L)] + s)
            cnt_local[seg0, pl.ds(0, L)] = cnt_local[seg0, pl.ds(0, L)] + float(G)

        @pl.when(jnp.logical_not(uniform))
        def _():
            _rows(sel, g, idxvs)

    def _block(j, carry):
        pltpu.make_async_copy(*_x_slices(j), sx).wait()
        pltpu.make_async_copy(*_i_slices(j), si).wait()

        @pl.when(j + 1 < NBLK)
        def _():
            pltpu.async_copy(*_x_slices(j + 1), sx)
            pltpu.async_copy(*_i_slices(j + 1), si)

        sel = lax.rem(j, 2)
        segb = _first_lane(idxbig[sel, pl.ds(0, L)])
        segl = _last_lane(idxbig[sel, pl.ds(BLK - L, L)])
        uniform_block = segb == segl

        @pl.when(uniform_block)
        def _():
            def _acc32(g, c):
                rb = sel * BLK + g * G
                parts = []
                for l in range(DH // L):
                    parts.append(_tree(
                        [xbig[rb + r, pl.ds(l * L, L)] for r in range(G)]))
                return tuple(c[l] + parts[l] for l in range(DH // L))
            tot = lax.fori_loop(0, BLK // G,
                                _acc32, tuple([zero16] * (DH // L)))
            for l in range(DH // L):
                acc_local[segb, pl.ds(l * L, L)] = (
                    acc_local[segb, pl.ds(l * L, L)] + tot[l])
            cnt_local[segb, pl.ds(0, L)] = (
                cnt_local[segb, pl.ds(0, L)] + float(BLK))

        @pl.when(jnp.logical_not(uniform_block))
        def _():
            def _g(g, c):
                _group(sel, g)
                return c
            lax.fori_loop(0, BLK // G, _g, 0)
        return carry
    pltpu.async_copy(*_x_slices(0), sx)
    pltpu.async_copy(*_i_slices(0), si)
    lax.fori_loop(0, NBLK, _block, 0)

    @pl.when(sid == NS - 1)
    def _():
        pltpu.sync_copy(x_hbm.at[pl.ds(TC_END, TAIL), pl.ds(col0, DH)],
                        xbig.at[pl.ds(0, TAIL)])
        pltpu.sync_copy(b_hbm.at[pl.ds(TC_END, TAIL)],
                        idxbig.at[0, pl.ds(0, TAIL)])
        def _g(g, c):
            _group(0, g)
            return c
        lax.fori_loop(0, TAIL // G, _g, 0)

    plsc.subcore_barrier()

    idrow = idbuf.at[0]
    pltpu.sync_copy(acc_local, acc_sh.at[idrow], add=True)
    pltpu.sync_copy(cnt_local, cnt_sh.at[idrow], add=True)

    plsc.subcore_barrier()

    @pl.when(sid < ACC_ROWS // 16)
    def _():
        r0 = sid * 16
        pltpu.sync_copy(acc_sh.at[pl.ds(r0, 16)],
                        sum_hbm.at[pl.ds(r0, 16), pl.ds(col0, DH)])
        @pl.when(cid == 0)
        def _():
            pltpu.sync_copy(cnt_sh.at[pl.ds(r0, 16)], cnt_hbm.at[pl.ds(r0, 16)])


_mesh = plsc.VectorSubcoreMesh(core_axis_name="c", subcore_axis_name="s",
                               num_cores=NC, num_subcores=NS)

_pool_sc = pl.kernel(
    _pool_body,
    out_type=(jax.ShapeDtypeStruct((ACC_ROWS, D), jnp.float32),
              jax.ShapeDtypeStruct((ACC_ROWS, 16), jnp.float32)),
    mesh=_mesh,
    scratch_types=[
        pltpu.VMEM((2 * BLK, DH), jnp.float32),
        pltpu.VMEM((2, 576), jnp.int32),
        pltpu.VMEM((1, ACC_ROWS), jnp.int32),
        pltpu.VMEM((16, DH), jnp.float32),
        pltpu.VMEM((ACC_ROWS, DH), jnp.float32),
        pltpu.VMEM((ACC_ROWS, 16), jnp.float32),
        pltpu.VMEM_SHARED((ACC_ROWS, DH), jnp.float32),
        pltpu.VMEM_SHARED((ACC_ROWS, 16), jnp.float32),
        pltpu.SemaphoreType.DMA,
        pltpu.SemaphoreType.DMA,
    ],
    compiler_params=pltpu.CompilerParams(use_tc_tiling_on_sc=False,
                                         needs_layout_passes=False),
)


def _tc_body(x_ref, b_ref, sum_ref, cnt_ref):
    i = pl.program_id(0)
    b = b_ref[0, 0, :]
    seg_ids = lax.broadcasted_iota(jnp.int32, (S, B_TC), 0)
    oh = (b[None, :] == seg_ids).astype(jnp.float32)
    part = lax.dot_general(oh, x_ref[...], (((1,), (0,)), ((), ())),
                           preferred_element_type=jnp.float32)
    pcnt = lax.dot_general(oh, jnp.ones((B_TC, 8), jnp.float32),
                           (((1,), (0,)), ((), ())),
                           preferred_element_type=jnp.float32)

    @pl.when(i == 0)
    def _():
        sum_ref[...] = part
        cnt_ref[...] = pcnt

    @pl.when(i > 0)
    def _():
        sum_ref[...] = sum_ref[...] + part
        cnt_ref[...] = cnt_ref[...] + pcnt


_pool_tc = pl.pallas_call(
    _tc_body,
    grid=(TC_BLOCKS,),
    in_specs=[
        pl.BlockSpec((B_TC, D), lambda i: (N_SC // B_TC + i, 0)),
        pl.BlockSpec((1, 1, B_TC), lambda i: (i, 0, 0)),
    ],
    out_specs=[
        pl.BlockSpec((S, D), lambda i: (0, 0)),
        pl.BlockSpec((S, 8), lambda i: (0, 0)),
    ],
    out_shape=[
        jax.ShapeDtypeStruct((S, D), jnp.float32),
        jax.ShapeDtypeStruct((S, 8), jnp.float32),
    ],
)


def _comb_body(ssc_ref, csc_ref, stc_ref, ctc_ref, out_ref):
    c = csc_ref[...][0:S, 0:1] + ctc_ref[...][0:S, 0:1]
    out_ref[...] = ((ssc_ref[...][0:S, :] + stc_ref[...])
                    / jnp.maximum(c, 1.0))


_combine = pl.pallas_call(
    _comb_body,
    out_shape=jax.ShapeDtypeStruct((S, D), jnp.float32),
)


def kernel(x, batch):
    b32 = batch.astype(jnp.int32)
    sum_sc, cnt_sc = _pool_sc(x, b32)
    b_tc = b32[N_SC:TC_END].reshape(TC_BLOCKS, 1, B_TC)
    sum_tc, cnt_tc = _pool_tc(x, b_tc)
    return _combine(sum_sc, cnt_sc, sum_tc, cnt_tc)


# speedup vs baseline: 1.8173x; 1.1003x over previous
import jax
import jax.numpy as jnp
from jax import lax
from jax.experimental import pallas as pl
from jax.experimental.pallas import tpu as pltpu
from jax.experimental.pallas import tpu_sc as plsc

N = 100000
D = 128
S = 64
NC = 2
NS = 16
L = 16
DH = D // NC
BLK = 448
Q = 3136
N_SC = NS * Q
NBLK = Q // BLK
G = 32
ACC_ROWS = 80
B_TC = 1792
TC_END = 98560
TC_BLOCKS = (TC_END - N_SC) // B_TC
TAILS = ((NS - 2, TC_END, 704), (NS - 1, TC_END + 704, 736))


def _first_lane(v):
    return lax.squeeze(lax.slice(v, (0,), (1,)), (0,))


def _last_lane(v):
    return lax.squeeze(lax.slice(v, (L - 1,), (L,)), (0,))


def _tree(vs):
    while len(vs) > 1:
        vs = [vs[i] + vs[i + 1] for i in range(0, len(vs) - 1, 2)] \
             + ([vs[-1]] if len(vs) % 2 else [])
    return vs[0]


def _pool_body(x_hbm, b_hbm, sum_hbm, cnt_hbm,
               xbig, idxbig, idbuf, zbuf,
               acc_local, cnt_local, acc_sh, cnt_sh, sx, si):
    cid = lax.axis_index("c")
    sid = lax.axis_index("s")
    col0 = cid * DH
    base0 = sid * Q

    zero16 = jnp.zeros((L,), jnp.float32)

    def _zrow(r, carry):
        for l in range(DH // L):
            acc_local[r, pl.ds(l * L, L)] = zero16
        cnt_local[r, pl.ds(0, L)] = zero16
        return carry
    lax.fori_loop(0, ACC_ROWS, _zrow, 0)

    iota16 = lax.iota(jnp.int32, 16)
    for k in range(ACC_ROWS // 16):
        idbuf[0, pl.ds(k * 16, 16)] = iota16 + (k * 16)

    for r in range(16):
        for l in range(DH // L):
            zbuf[r, pl.ds(l * L, L)] = zero16
    @pl.when(sid == 0)
    def _():
        for r0 in range(0, ACC_ROWS, 16):
            pltpu.sync_copy(zbuf, acc_sh.at[pl.ds(r0, 16)])
            pltpu.sync_copy(zbuf.at[:, pl.ds(0, 16)], cnt_sh.at[pl.ds(r0, 16)])

    def _x_slices(j):
        base = base0 + j * BLK
        sel = lax.rem(j, 2)
        return (x_hbm.at[pl.ds(base, BLK), pl.ds(col0, DH)],
                xbig.at[pl.ds(sel * BLK, BLK)])

    def _i_slices(j):
        base = base0 + j * BLK
        sel = lax.rem(j, 2)
        return b_hbm.at[pl.ds(base, BLK)], idxbig.at[sel, pl.ds(0, BLK)]

    def _rows(sel, g, idxvs):
        rb = sel * BLK + g * G
        for v in range(G // L):
            for r in range(L):
                sr = lax.squeeze(lax.slice(idxvs[v], (r,), (r + 1,)), (0,))
                for l in range(DH // L):
                    acc_local[sr, pl.ds(l * L, L)] = (
                        acc_local[sr, pl.ds(l * L, L)]
                        + xbig[rb + v * L + r, pl.ds(l * L, L)])
                cnt_local[sr, pl.ds(0, L)] = cnt_local[sr, pl.ds(0, L)] + 1.0

    def _group(sel, g):
        rb = sel * BLK + g * G
        idxvs = [idxbig[sel, pl.ds(g * G + v * L, L)] for v in range(G // L)]
        seg0 = _first_lane(idxvs[0])
        uniform = seg0 == _last_lane(idxvs[-1])

        @pl.when(uniform)
        def _():
            for l in range(DH // L):
                s = _tree([xbig[rb + r, pl.ds(l * L, L)] for r in range(G)])
                acc_local[seg0, pl.ds(l * L, L)] = (
                    acc_local[seg0, pl.ds(l * L, L)] + s)
            cnt_local[seg0, pl.ds(0, L)] = cnt_local[seg0, pl.ds(0, L)] + float(G)

        @pl.when(jnp.logical_not(uniform))
        def _():
            _rows(sel, g, idxvs)

    def _block(j, carry):
        pltpu.make_async_copy(*_x_slices(j), sx).wait()
        pltpu.make_async_copy(*_i_slices(j), si).wait()

        @pl.when(j + 1 < NBLK)
        def _():
            pltpu.async_copy(*_x_slices(j + 1), sx)
            pltpu.async_copy(*_i_slices(j + 1), si)

        sel = lax.rem(j, 2)
        segb = _first_lane(idxbig[sel, pl.ds(0, L)])
        segl = _last_lane(idxbig[sel, pl.ds(BLK - L, L)])
        uniform_block = segb == segl

        @pl.when(uniform_block)
        def _():
            def _acc32(g, c):
                rb = sel * BLK + g * G
                parts = []
                for l in range(DH // L):
                    parts.append(_tree(
                        [xbig[rb + r, pl.ds(l * L, L)] for r in range(G)]))
                return tuple(c[l] + parts[l] for l in range(DH // L))
            tot = lax.fori_loop(0, BLK // G,
                                _acc32, tuple([zero16] * (DH // L)))
            for l in range(DH // L):
                acc_local[segb, pl.ds(l * L, L)] = (
                    acc_local[segb, pl.ds(l * L, L)] + tot[l])
            cnt_local[segb, pl.ds(0, L)] = (
                cnt_local[segb, pl.ds(0, L)] + float(BLK))

        @pl.when(jnp.logical_not(uniform_block))
        def _():
            def _g(g, c):
                _group(sel, g)
                return c
            lax.fori_loop(0, BLK // G, _g, 0)
        return carry
    pltpu.async_copy(*_x_slices(0), sx)
    pltpu.async_copy(*_i_slices(0), si)
    lax.fori_loop(0, NBLK, _block, 0)

    for t_sid, t_off, t_n in TAILS:
        @pl.when(sid == t_sid)
        def _(t_off=t_off, t_n=t_n):
            pltpu.sync_copy(x_hbm.at[pl.ds(t_off, t_n), pl.ds(col0, DH)],
                            xbig.at[pl.ds(0, t_n)])
            pltpu.sync_copy(b_hbm.at[pl.ds(t_off, t_n)],
                            idxbig.at[0, pl.ds(0, t_n)])
            def _g(g, c):
                _group(0, g)
                return c
            lax.fori_loop(0, t_n // G, _g, 0)

    plsc.subcore_barrier()

    idrow = idbuf.at[0]
    pltpu.sync_copy(acc_local, acc_sh.at[idrow], add=True)
    pltpu.sync_copy(cnt_local, cnt_sh.at[idrow], add=True)

    plsc.subcore_barrier()

    @pl.when(sid < ACC_ROWS // 16)
    def _():
        r0 = sid * 16
        pltpu.sync_copy(acc_sh.at[pl.ds(r0, 16)],
                        sum_hbm.at[pl.ds(r0, 16), pl.ds(col0, DH)])
        @pl.when(cid == 0)
        def _():
            pltpu.sync_copy(cnt_sh.at[pl.ds(r0, 16)], cnt_hbm.at[pl.ds(r0, 16)])


_mesh = plsc.VectorSubcoreMesh(core_axis_name="c", subcore_axis_name="s",
                               num_cores=NC, num_subcores=NS)

_pool_sc = pl.kernel(
    _pool_body,
    out_type=(jax.ShapeDtypeStruct((ACC_ROWS, D), jnp.float32),
              jax.ShapeDtypeStruct((ACC_ROWS, 16), jnp.float32)),
    mesh=_mesh,
    scratch_types=[
        pltpu.VMEM((2 * BLK, DH), jnp.float32),
        pltpu.VMEM((2, 768), jnp.int32),
        pltpu.VMEM((1, ACC_ROWS), jnp.int32),
        pltpu.VMEM((16, DH), jnp.float32),
        pltpu.VMEM((ACC_ROWS, DH), jnp.float32),
        pltpu.VMEM((ACC_ROWS, 16), jnp.float32),
        pltpu.VMEM_SHARED((ACC_ROWS, DH), jnp.float32),
        pltpu.VMEM_SHARED((ACC_ROWS, 16), jnp.float32),
        pltpu.SemaphoreType.DMA,
        pltpu.SemaphoreType.DMA,
    ],
    compiler_params=pltpu.CompilerParams(use_tc_tiling_on_sc=False,
                                         needs_layout_passes=False),
)


def _tc_body(x_ref, b_ref, sum_ref, cnt_ref):
    i = pl.program_id(0)
    b = b_ref[0, 0, :]
    seg_ids = lax.broadcasted_iota(jnp.int32, (S, B_TC), 0)
    oh = (b[None, :] == seg_ids).astype(jnp.float32)
    part = lax.dot_general(oh, x_ref[...], (((1,), (0,)), ((), ())),
                           preferred_element_type=jnp.float32)
    pcnt = lax.dot_general(oh, jnp.ones((B_TC, 8), jnp.float32),
                           (((1,), (0,)), ((), ())),
                           preferred_element_type=jnp.float32)

    @pl.when(i == 0)
    def _():
        sum_ref[...] = part
        cnt_ref[...] = pcnt

    @pl.when(i > 0)
    def _():
        sum_ref[...] = sum_ref[...] + part
        cnt_ref[...] = cnt_ref[...] + pcnt


_pool_tc = pl.pallas_call(
    _tc_body,
    grid=(TC_BLOCKS,),
    in_specs=[
        pl.BlockSpec((B_TC, D), lambda i: (N_SC // B_TC + i, 0)),
        pl.BlockSpec((1, 1, B_TC), lambda i: (i, 0, 0)),
    ],
    out_specs=[
        pl.BlockSpec((S, D), lambda i: (0, 0)),
        pl.BlockSpec((S, 8), lambda i: (0, 0)),
    ],
    out_shape=[
        jax.ShapeDtypeStruct((S, D), jnp.float32),
        jax.ShapeDtypeStruct((S, 8), jnp.float32),
    ],
)


def _comb_body(ssc_ref, csc_ref, stc_ref, ctc_ref, out_ref):
    c = csc_ref[...][0:S, 0:1] + ctc_ref[...][0:S, 0:1]
    out_ref[...] = ((ssc_ref[...][0:S, :] + stc_ref[...])
                    / jnp.maximum(c, 1.0))


_combine = pl.pallas_call(
    _comb_body,
    out_shape=jax.ShapeDtypeStruct((S, D), jnp.float32),
)


def kernel(x, batch):
    b32 = batch.astype(jnp.int32)
    sum_sc, cnt_sc = _pool_sc(x, b32)
    b_tc = b32[N_SC:TC_END].reshape(TC_BLOCKS, 1, B_TC)
    sum_tc, cnt_tc = _pool_tc(x, b_tc)
    return _combine(sum_sc, cnt_sc, sum_tc, cnt_tc)


# speedup vs baseline: 2.0584x; 1.1327x over previous
import jax
import jax.numpy as jnp
from jax import lax
from jax.experimental import pallas as pl
from jax.experimental.pallas import tpu as pltpu
from jax.experimental.pallas import tpu_sc as plsc

N = 100000
D = 128
S = 64
NC = 2
NS = 16
L = 16
DH = D // NC
BLK = 448
Q = 2688
N_SC = NS * Q
NBLK = Q // BLK
G = 32
ACC_ROWS = 80
B_TC = 1792
TC_END = 98560
TC_BLOCKS = (TC_END - N_SC) // B_TC
TAIL_PER_TILE = 96


def _first_lane(v):
    return lax.squeeze(lax.slice(v, (0,), (1,)), (0,))


def _last_lane(v):
    return lax.squeeze(lax.slice(v, (L - 1,), (L,)), (0,))


def _tree(vs):
    while len(vs) > 1:
        vs = [vs[i] + vs[i + 1] for i in range(0, len(vs) - 1, 2)] \
             + ([vs[-1]] if len(vs) % 2 else [])
    return vs[0]


def _pool_body(x_hbm, b_hbm, sum_hbm, cnt_hbm,
               xbig, idxbig, idbuf, zbuf,
               acc_local, cnt_local, acc_sh, cnt_sh, sx, si):
    cid = lax.axis_index("c")
    sid = lax.axis_index("s")
    col0 = cid * DH
    base0 = sid * Q

    zero16 = jnp.zeros((L,), jnp.float32)

    def _zrow(r, carry):
        for l in range(DH // L):
            acc_local[r, pl.ds(l * L, L)] = zero16
        cnt_local[r, pl.ds(0, L)] = zero16
        return carry
    lax.fori_loop(0, ACC_ROWS, _zrow, 0)

    iota16 = lax.iota(jnp.int32, 16)
    for k in range(ACC_ROWS // 16):
        idbuf[0, pl.ds(k * 16, 16)] = iota16 + (k * 16)

    for r in range(16):
        for l in range(DH // L):
            zbuf[r, pl.ds(l * L, L)] = zero16
    @pl.when(sid == 0)
    def _():
        for r0 in range(0, ACC_ROWS, 16):
            pltpu.sync_copy(zbuf, acc_sh.at[pl.ds(r0, 16)])
            pltpu.sync_copy(zbuf.at[:, pl.ds(0, 16)], cnt_sh.at[pl.ds(r0, 16)])

    def _x_slices(j):
        base = base0 + j * BLK
        sel = lax.rem(j, 2)
        return (x_hbm.at[pl.ds(base, BLK), pl.ds(col0, DH)],
                xbig.at[pl.ds(sel * BLK, BLK)])

    def _i_slices(j):
        base = base0 + j * BLK
        sel = lax.rem(j, 2)
        return b_hbm.at[pl.ds(base, BLK)], idxbig.at[sel, pl.ds(0, BLK)]

    def _rows(sel, g, idxvs):
        rb = sel * BLK + g * G
        for v in range(G // L):
            for r in range(L):
                sr = lax.squeeze(lax.slice(idxvs[v], (r,), (r + 1,)), (0,))
                for l in range(DH // L):
                    acc_local[sr, pl.ds(l * L, L)] = (
                        acc_local[sr, pl.ds(l * L, L)]
                        + xbig[rb + v * L + r, pl.ds(l * L, L)])
                cnt_local[sr, pl.ds(0, L)] = cnt_local[sr, pl.ds(0, L)] + 1.0

    def _group(sel, g):
        rb = sel * BLK + g * G
        idxvs = [idxbig[sel, pl.ds(g * G + v * L, L)] for v in range(G // L)]
        seg0 = _first_lane(idxvs[0])
        uniform = seg0 == _last_lane(idxvs[-1])

        @pl.when(uniform)
        def _():
            for l in range(DH // L):
                s = _tree([xbig[rb + r, pl.ds(l * L, L)] for r in range(G)])
                acc_local[seg0, pl.ds(l * L, L)] = (
                    acc_local[seg0, pl.ds(l * L, L)] + s)
            cnt_local[seg0, pl.ds(0, L)] = cnt_local[seg0, pl.ds(0, L)] + float(G)

        @pl.when(jnp.logical_not(uniform))
        def _():
            _rows(sel, g, idxvs)

    def _block(j, carry):
        pltpu.make_async_copy(*_x_slices(j), sx).wait()
        pltpu.make_async_copy(*_i_slices(j), si).wait()

        @pl.when(j + 1 < NBLK)
        def _():
            pltpu.async_copy(*_x_slices(j + 1), sx)
            pltpu.async_copy(*_i_slices(j + 1), si)

        sel = lax.rem(j, 2)
        segb = _first_lane(idxbig[sel, pl.ds(0, L)])
        segl = _last_lane(idxbig[sel, pl.ds(BLK - L, L)])
        uniform_block = segb == segl

        @pl.when(uniform_block)
        def _():
            def _acc32(g, c):
                rb = sel * BLK + g * G
                parts = []
                for l in range(DH // L):
                    parts.append(_tree(
                        [xbig[rb + r, pl.ds(l * L, L)] for r in range(G)]))
                return tuple(c[l] + parts[l] for l in range(DH // L))
            tot = lax.fori_loop(0, BLK // G,
                                _acc32, tuple([zero16] * (DH // L)))
            for l in range(DH // L):
                acc_local[segb, pl.ds(l * L, L)] = (
                    acc_local[segb, pl.ds(l * L, L)] + tot[l])
            cnt_local[segb, pl.ds(0, L)] = (
                cnt_local[segb, pl.ds(0, L)] + float(BLK))

        @pl.when(jnp.logical_not(uniform_block))
        def _():
            def _g(g, c):
                _group(sel, g)
                return c
            lax.fori_loop(0, BLK // G, _g, 0)
        return carry
    pltpu.async_copy(*_x_slices(0), sx)
    pltpu.async_copy(*_i_slices(0), si)
    lax.fori_loop(0, NBLK, _block, 0)

    @pl.when(sid < NS - 1)
    def _():
        t_off = TC_END + sid * TAIL_PER_TILE
        pltpu.sync_copy(x_hbm.at[pl.ds(t_off, TAIL_PER_TILE), pl.ds(col0, DH)],
                        xbig.at[pl.ds(0, TAIL_PER_TILE)])
        pltpu.sync_copy(b_hbm.at[pl.ds(t_off, TAIL_PER_TILE)],
                        idxbig.at[0, pl.ds(0, TAIL_PER_TILE)])
        def _g(g, c):
            _group(0, g)
            return c
        lax.fori_loop(0, TAIL_PER_TILE // G, _g, 0)

    plsc.subcore_barrier()

    idrow = idbuf.at[0]
    pltpu.sync_copy(acc_local, acc_sh.at[idrow], add=True)
    pltpu.sync_copy(cnt_local, cnt_sh.at[idrow], add=True)

    plsc.subcore_barrier()

    @pl.when(sid < ACC_ROWS // 16)
    def _():
        r0 = sid * 16
        pltpu.sync_copy(acc_sh.at[pl.ds(r0, 16)],
                        sum_hbm.at[pl.ds(r0, 16), pl.ds(col0, DH)])
        @pl.when(cid == 0)
        def _():
            pltpu.sync_copy(cnt_sh.at[pl.ds(r0, 16)], cnt_hbm.at[pl.ds(r0, 16)])


_mesh = plsc.VectorSubcoreMesh(core_axis_name="c", subcore_axis_name="s",
                               num_cores=NC, num_subcores=NS)

_pool_sc = pl.kernel(
    _pool_body,
    out_type=(jax.ShapeDtypeStruct((ACC_ROWS, D), jnp.float32),
              jax.ShapeDtypeStruct((ACC_ROWS, 16), jnp.float32)),
    mesh=_mesh,
    scratch_types=[
        pltpu.VMEM((2 * BLK, DH), jnp.float32),
        pltpu.VMEM((2, 768), jnp.int32),
        pltpu.VMEM((1, ACC_ROWS), jnp.int32),
        pltpu.VMEM((16, DH), jnp.float32),
        pltpu.VMEM((ACC_ROWS, DH), jnp.float32),
        pltpu.VMEM((ACC_ROWS, 16), jnp.float32),
        pltpu.VMEM_SHARED((ACC_ROWS, DH), jnp.float32),
        pltpu.VMEM_SHARED((ACC_ROWS, 16), jnp.float32),
        pltpu.SemaphoreType.DMA,
        pltpu.SemaphoreType.DMA,
    ],
    compiler_params=pltpu.CompilerParams(use_tc_tiling_on_sc=False,
                                         needs_layout_passes=False),
)


def _tc_body(x_ref, b_ref, sum_ref, cnt_ref):
    i = pl.program_id(0)
    b = b_ref[0, 0, :]
    seg_ids = lax.broadcasted_iota(jnp.int32, (S, B_TC), 0)
    oh = (b[None, :] == seg_ids).astype(jnp.float32)
    part = lax.dot_general(oh, x_ref[...], (((1,), (0,)), ((), ())),
                           preferred_element_type=jnp.float32)
    pcnt = lax.dot_general(oh, jnp.ones((B_TC, 8), jnp.float32),
                           (((1,), (0,)), ((), ())),
                           preferred_element_type=jnp.float32)

    @pl.when(i == 0)
    def _():
        sum_ref[...] = part
        cnt_ref[...] = pcnt

    @pl.when(i > 0)
    def _():
        sum_ref[...] = sum_ref[...] + part
        cnt_ref[...] = cnt_ref[...] + pcnt


_pool_tc = pl.pallas_call(
    _tc_body,
    grid=(TC_BLOCKS,),
    in_specs=[
        pl.BlockSpec((B_TC, D), lambda i: (N_SC // B_TC + i, 0)),
        pl.BlockSpec((1, 1, B_TC), lambda i: (i, 0, 0)),
    ],
    out_specs=[
        pl.BlockSpec((S, D), lambda i: (0, 0)),
        pl.BlockSpec((S, 8), lambda i: (0, 0)),
    ],
    out_shape=[
        jax.ShapeDtypeStruct((S, D), jnp.float32),
        jax.ShapeDtypeStruct((S, 8), jnp.float32),
    ],
)


def _comb_body(ssc_ref, csc_ref, stc_ref, ctc_ref, out_ref):
    c = csc_ref[...][0:S, 0:1] + ctc_ref[...][0:S, 0:1]
    out_ref[...] = ((ssc_ref[...][0:S, :] + stc_ref[...])
                    / jnp.maximum(c, 1.0))


_combine = pl.pallas_call(
    _comb_body,
    out_shape=jax.ShapeDtypeStruct((S, D), jnp.float32),
)


def kernel(x, batch):
    b32 = batch.astype(jnp.int32)
    sum_sc, cnt_sc = _pool_sc(x, b32)
    b_tc = b32[N_SC:TC_END].reshape(TC_BLOCKS, 1, B_TC)
    sum_tc, cnt_tc = _pool_tc(x, b_tc)
    return _combine(sum_sc, cnt_sc, sum_tc, cnt_tc)
